# baseline (speedup 1.0000x reference)
import functools

import jax
import jax.numpy as jnp
from jax import lax
from jax.experimental import pallas as pl
from jax.experimental.pallas import tpu as pltpu
from jax.experimental.pallas import tpu_sc as plsc

N = 10000
D = 128
E = 320000
NC = 2
NS = 16
NW = NC * NS
CH = 80
NCH = (E // NW) // CH
NPAD = 10240
RPT = NPAD // NS

_mesh = plsc.VectorSubcoreMesh(
    core_axis_name="c", subcore_axis_name="s", num_cores=NC, num_subcores=NS
)


@functools.partial(
    pl.kernel,
    out_type=jax.ShapeDtypeStruct((NC, NPAD), jnp.float32),
    mesh=_mesh,
    scratch_types=[
        pltpu.VMEM((NCH, CH), jnp.int32),
        pltpu.VMEM((CH,), jnp.float32),
        pltpu.SemaphoreType.DMA,
        pltpu.VMEM_SHARED((NPAD,), jnp.float32),
    ],
)
def _sc_degree(ei_hbm, zero_hbm, out_hbm, dst_v, ones_v, sem, accum):
    c = lax.axis_index("c")
    s = lax.axis_index("s")
    wid = s * NC + c
    _offs = sorted(set(list(range(0, CH - 15, 16)) + [CH - 16]))
    for _o in _offs:
        ones_v[pl.ds(_o, 16)] = jnp.ones((16,), jnp.float32)
    pltpu.sync_copy(ei_hbm.at[1, wid], dst_v)
    pltpu.sync_copy(zero_hbm, accum.at[pl.ds(s * RPT, RPT)])
    plsc.subcore_barrier()

    LAG = 4

    def body(j, carry):
        pltpu.async_copy(ones_v, accum.at[dst_v.at[j]], sem, add=True)

        @pl.when(j >= LAG)
        def _():
            pltpu.make_async_copy(ones_v, accum.at[dst_v.at[j - LAG]], sem).wait()
        return carry

    lax.fori_loop(0, NCH, body, 0)
    for _t in range(LAG):
        pltpu.make_async_copy(ones_v, accum.at[dst_v.at[NCH - LAG + _t]], sem).wait()
    plsc.subcore_barrier()
    pltpu.sync_copy(accum.at[pl.ds(s * RPT, RPT)], out_hbm.at[c, pl.ds(s * RPT, RPT)])


@functools.partial(
    pl.kernel,
    out_type=jax.ShapeDtypeStruct((NC, NPAD, D), jnp.float32),
    mesh=_mesh,
    scratch_types=[
        pltpu.VMEM((64, CH), jnp.int32),
        pltpu.VMEM((64, CH), jnp.int32),
        pltpu.VMEM((CH, D), jnp.float32),
        pltpu.VMEM((CH, D), jnp.float32),
        pltpu.VMEM((CH, D), jnp.float32),
        pltpu.SemaphoreType.DMA,
        pltpu.SemaphoreType.DMA,
        pltpu.VMEM_SHARED((NPAD, D), jnp.float32),
    ],
)
def _sc_aggregate(ei_hbm, g_hbm, zero_hbm, out_hbm,
                  src_v, dst_v, gbuf0, gbuf1, gbuf2, sem_g, sem_s, accum):
    c = lax.axis_index("c")
    s = lax.axis_index("s")
    wid = s * NC + c
    for t in range(RPT // 128):
        _r = s * RPT + t * 128
        pltpu.async_copy(zero_hbm.at[pl.ds(_r, 128)], accum.at[pl.ds(_r, 128)], sem_g)
    pltpu.sync_copy(ei_hbm.at[0, wid, pl.ds(0, 64)], src_v)
    pltpu.sync_copy(ei_hbm.at[1, wid, pl.ds(0, 64)], dst_v)
    for t in range(RPT // 128):
        _r = s * RPT + t * 128
        pltpu.make_async_copy(zero_hbm.at[pl.ds(_r, 128)], accum.at[pl.ds(_r, 128)], sem_g).wait()
    plsc.subcore_barrier()

    bufs = (gbuf0, gbuf1, gbuf2)

    def _wait_scatter(j, b):
        pltpu.make_async_copy(bufs[b], accum.at[dst_v.at[j]], sem_s).wait()

    for base, rows in ((0, 64), (64, NCH - 64)):
        if base > 0:
            pltpu.sync_copy(ei_hbm.at[0, wid, pl.ds(base, rows)], src_v.at[pl.ds(0, rows)])
            pltpu.sync_copy(ei_hbm.at[1, wid, pl.ds(base, rows)], dst_v.at[pl.ds(0, rows)])
        pltpu.async_copy(g_hbm.at[src_v.at[0]], bufs[0], sem_g)
        pltpu.async_copy(g_hbm.at[src_v.at[1]], bufs[1], sem_g)

        def body(k, carry):
            for b in range(3):
                j = 3 * k + b
                pltpu.make_async_copy(g_hbm.at[src_v.at[j]], bufs[b], sem_g).wait()
                pltpu.async_copy(bufs[b], accum.at[dst_v.at[j]], sem_s, add=True)

                @pl.when(j >= 1)
                def _():
                    _wait_scatter(j - 1, (b - 1) % 3)

                @pl.when(j + 2 < rows)
                def _():
                    pltpu.async_copy(g_hbm.at[src_v.at[j + 2]], bufs[(b + 2) % 3], sem_g)
            return carry

        lax.fori_loop(0, rows // 3, body, 0)
        for j in range(3 * (rows // 3), rows):
            b = j % 3
            pltpu.make_async_copy(g_hbm.at[src_v.at[j]], bufs[b], sem_g).wait()
            pltpu.async_copy(bufs[b], accum.at[dst_v.at[j]], sem_s, add=True)
            if j >= 1:
                _wait_scatter(j - 1, (b - 1) % 3)
            if j + 2 < rows:
                pltpu.async_copy(g_hbm.at[src_v.at[j + 2]], bufs[(b + 2) % 3], sem_g)
        _wait_scatter(rows - 1, (rows - 1) % 3)
    plsc.subcore_barrier()
    for t in range(RPT // 128):
        _r = s * RPT + t * 128
        pltpu.async_copy(accum.at[pl.ds(_r, 128)], out_hbm.at[c, pl.ds(_r, 128)], sem_g)
    for t in range(RPT // 128):
        _r = s * RPT + t * 128
        pltpu.make_async_copy(accum.at[pl.ds(_r, 128)], out_hbm.at[c, pl.ds(_r, 128)], sem_g).wait()


BR = 10000
GRID = N // BR


def _dinv_block(dg_ref):
    deg = dg_ref[0, 0] + dg_ref[0, 1] + 1.0
    return jax.lax.rsqrt(deg)[:, None]


def _tc_g_body(x_ref, w_ref, dg_ref, o_ref):
    h = jnp.dot(x_ref[...], w_ref[...], preferred_element_type=jnp.float32)
    o_ref[...] = h * _dinv_block(dg_ref)


def _tc_mid_body(p_ref, g_ref, dg_ref, b_ref, w_ref, o_ref):
    dinv = _dinv_block(dg_ref)
    z = (p_ref[0] + p_ref[1] + g_ref[...]) * dinv + b_ref[...]
    a = jax.nn.gelu(z)
    h2 = jnp.dot(a, w_ref[...], preferred_element_type=jnp.float32)
    o_ref[...] = h2 * dinv


def _tc_out_body(p_ref, g_ref, dg_ref, b_ref, o_ref):
    z = (p_ref[0] + p_ref[1] + g_ref[...]) * _dinv_block(dg_ref) + b_ref[...]
    m = jnp.max(z, axis=-1, keepdims=True)
    e = jnp.exp(z - m)
    o_ref[...] = e / jnp.sum(e, axis=-1, keepdims=True)


_DG_SPEC = pl.BlockSpec((1, NC, BR), lambda i: (i, 0, 0))


def _tc_g(x, W, degp):
    return pl.pallas_call(
        _tc_g_body,
        grid=(GRID,),
        in_specs=[
            pl.BlockSpec((BR, D), lambda i: (i, 0)),
            pl.BlockSpec((D, D), lambda i: (0, 0)),
            _DG_SPEC,
        ],
        out_specs=pl.BlockSpec((BR, D), lambda i: (i, 0)),
        out_shape=jax.ShapeDtypeStruct((N, D), jnp.float32),
    )(x, W, degp)


def _tc_mid(p, g, degp, b_row, W):
    return pl.pallas_call(
        _tc_mid_body,
        grid=(GRID,),
        in_specs=[
            pl.BlockSpec((NC, BR, D), lambda i: (0, i, 0)),
            pl.BlockSpec((BR, D), lambda i: (i, 0)),
            _DG_SPEC,
            pl.BlockSpec((1, D), lambda i: (0, 0)),
            pl.BlockSpec((D, D), lambda i: (0, 0)),
        ],
        out_specs=pl.BlockSpec((BR, D), lambda i: (i, 0)),
        out_shape=jax.ShapeDtypeStruct((N, D), jnp.float32),
    )(p, g, degp, b_row, W)


def _tc_out(p, g, degp, b_row):
    return pl.pallas_call(
        _tc_out_body,
        grid=(GRID,),
        in_specs=[
            pl.BlockSpec((NC, BR, D), lambda i: (0, i, 0)),
            pl.BlockSpec((BR, D), lambda i: (i, 0)),
            _DG_SPEC,
            pl.BlockSpec((1, D), lambda i: (0, 0)),
        ],
        out_specs=pl.BlockSpec((BR, D), lambda i: (i, 0)),
        out_shape=jax.ShapeDtypeStruct((N, D), jnp.float32),
    )(p, g, degp, b_row)


def kernel(x, edge_index, W1, b1, W2, b2):
    ei4 = edge_index.reshape(2, NW, NCH, CH)
    zero2 = jnp.zeros((NPAD, D), jnp.float32)
    zero1 = jnp.zeros((RPT,), jnp.float32)

    degp = _sc_degree(ei4, zero1)[:, :N].reshape(NC, GRID, BR).transpose(1, 0, 2)
    g1 = _tc_g(x, W1, degp)
    p1 = _sc_aggregate(ei4, g1, zero2)
    g2 = _tc_mid(p1, g1, degp, b1.reshape(1, D), W2)
    p2 = _sc_aggregate(ei4, g2, zero2)
    return _tc_out(p2, g2, degp, b2.reshape(1, D))


# speedup vs baseline: 1.0180x; 1.0180x over previous
import functools

import jax
import jax.numpy as jnp
from jax import lax
from jax.experimental import pallas as pl
from jax.experimental.pallas import tpu as pltpu
from jax.experimental.pallas import tpu_sc as plsc

N = 10000
D = 128
E = 320000
NC = 2
NS = 16
NW = NC * NS
CH = 80
NCH = (E // NW) // CH
NPAD = 10240
RPT = NPAD // NS

_mesh = plsc.VectorSubcoreMesh(
    core_axis_name="c", subcore_axis_name="s", num_cores=NC, num_subcores=NS
)


@functools.partial(
    pl.kernel,
    out_type=jax.ShapeDtypeStruct((NC, NPAD), jnp.float32),
    mesh=_mesh,
    scratch_types=[
        pltpu.VMEM((NCH, CH), jnp.int32),
        pltpu.VMEM((CH,), jnp.float32),
        pltpu.SemaphoreType.DMA,
        pltpu.VMEM_SHARED((NPAD,), jnp.float32),
    ],
)
def _sc_degree(ei_hbm, zero_hbm, out_hbm, dst_v, ones_v, sem, accum):
    c = lax.axis_index("c")
    s = lax.axis_index("s")
    wid = s * NC + c
    _offs = sorted(set(list(range(0, CH - 15, 16)) + [CH - 16]))
    for _o in _offs:
        ones_v[pl.ds(_o, 16)] = jnp.ones((16,), jnp.float32)
    pltpu.sync_copy(ei_hbm.at[1, wid], dst_v)
    pltpu.sync_copy(zero_hbm, accum.at[pl.ds(s * RPT, RPT)])
    plsc.subcore_barrier()

    LAG = 4

    def body(j, carry):
        pltpu.async_copy(ones_v, accum.at[dst_v.at[j]], sem, add=True)

        @pl.when(j >= LAG)
        def _():
            pltpu.make_async_copy(ones_v, accum.at[dst_v.at[j - LAG]], sem).wait()
        return carry

    lax.fori_loop(0, NCH, body, 0)
    for _t in range(LAG):
        pltpu.make_async_copy(ones_v, accum.at[dst_v.at[NCH - LAG + _t]], sem).wait()
    plsc.subcore_barrier()
    pltpu.sync_copy(accum.at[pl.ds(s * RPT, RPT)], out_hbm.at[c, pl.ds(s * RPT, RPT)])


@functools.partial(
    pl.kernel,
    out_type=jax.ShapeDtypeStruct((NC, NPAD, D), jnp.float32),
    mesh=_mesh,
    scratch_types=[
        pltpu.VMEM((64, CH), jnp.int32),
        pltpu.VMEM((64, CH), jnp.int32),
        pltpu.VMEM((CH, D), jnp.float32),
        pltpu.VMEM((CH, D), jnp.float32),
        pltpu.VMEM((CH, D), jnp.float32),
        pltpu.SemaphoreType.DMA,
        pltpu.SemaphoreType.DMA,
        pltpu.VMEM_SHARED((NPAD, D), jnp.float32),
    ],
)
def _sc_aggregate(ei_hbm, g_hbm, zero_hbm, out_hbm,
                  src_v, dst_v, gbuf0, gbuf1, gbuf2, sem_g, sem_s, accum):
    c = lax.axis_index("c")
    s = lax.axis_index("s")
    wid = s * NC + c
    for t in range(RPT // 128):
        _r = s * RPT + t * 128
        pltpu.async_copy(zero_hbm.at[pl.ds(_r, 128)], accum.at[pl.ds(_r, 128)], sem_g)
    pltpu.sync_copy(ei_hbm.at[0, wid, pl.ds(0, 64)], src_v)
    pltpu.sync_copy(ei_hbm.at[1, wid, pl.ds(0, 64)], dst_v)
    for t in range(RPT // 128):
        _r = s * RPT + t * 128
        pltpu.make_async_copy(zero_hbm.at[pl.ds(_r, 128)], accum.at[pl.ds(_r, 128)], sem_g).wait()
    plsc.subcore_barrier()

    bufs = (gbuf0, gbuf1, gbuf2)

    def _wait_scatter(j, b):
        pltpu.make_async_copy(bufs[b], accum.at[dst_v.at[j]], sem_s).wait()

    for base, rows in ((0, 64), (64, NCH - 64)):
        if base > 0:
            pltpu.sync_copy(ei_hbm.at[0, wid, pl.ds(base, rows)], src_v.at[pl.ds(0, rows)])
            pltpu.sync_copy(ei_hbm.at[1, wid, pl.ds(base, rows)], dst_v.at[pl.ds(0, rows)])
        pltpu.async_copy(g_hbm.at[src_v.at[0]], bufs[0], sem_g)
        pltpu.async_copy(g_hbm.at[src_v.at[1]], bufs[1], sem_g)

        def body(k, carry):
            for b in range(3):
                j = 3 * k + b
                pltpu.make_async_copy(g_hbm.at[src_v.at[j]], bufs[b], sem_g).wait()
                pltpu.async_copy(bufs[b], accum.at[dst_v.at[j]], sem_s, add=True)

                @pl.when(j >= 1)
                def _():
                    _wait_scatter(j - 1, (b - 1) % 3)

                @pl.when(j + 2 < rows)
                def _():
                    pltpu.async_copy(g_hbm.at[src_v.at[j + 2]], bufs[(b + 2) % 3], sem_g)
            return carry

        lax.fori_loop(0, rows // 3, body, 0)
        for j in range(3 * (rows // 3), rows):
            b = j % 3
            pltpu.make_async_copy(g_hbm.at[src_v.at[j]], bufs[b], sem_g).wait()
            pltpu.async_copy(bufs[b], accum.at[dst_v.at[j]], sem_s, add=True)
            if j >= 1:
                _wait_scatter(j - 1, (b - 1) % 3)
            if j + 2 < rows:
                pltpu.async_copy(g_hbm.at[src_v.at[j + 2]], bufs[(b + 2) % 3], sem_g)
        _wait_scatter(rows - 1, (rows - 1) % 3)
    plsc.subcore_barrier()
    for t in range(RPT // 128):
        _r = s * RPT + t * 128
        pltpu.async_copy(accum.at[pl.ds(_r, 128)], out_hbm.at[c, pl.ds(_r, 128)], sem_g)
    for t in range(RPT // 128):
        _r = s * RPT + t * 128
        pltpu.make_async_copy(accum.at[pl.ds(_r, 128)], out_hbm.at[c, pl.ds(_r, 128)], sem_g).wait()


BR = 5000
GRID = N // BR


def _dinv_block(dg_ref):
    deg = dg_ref[0, 0] + dg_ref[0, 1] + 1.0
    return jax.lax.rsqrt(deg)[:, None]


def _tc_g_body(x_ref, w_ref, dg_ref, o_ref):
    h = jnp.dot(x_ref[...], w_ref[...], preferred_element_type=jnp.float32)
    o_ref[...] = h * _dinv_block(dg_ref)


def _tc_mid_body(p_ref, g_ref, dg_ref, b_ref, w_ref, o_ref):
    dinv = _dinv_block(dg_ref)
    z = (p_ref[0] + p_ref[1] + g_ref[...]) * dinv + b_ref[...]
    a = jax.nn.gelu(z)
    h2 = jnp.dot(a, w_ref[...], preferred_element_type=jnp.float32)
    o_ref[...] = h2 * dinv


def _tc_out_body(p_ref, g_ref, dg_ref, b_ref, o_ref):
    z = (p_ref[0] + p_ref[1] + g_ref[...]) * _dinv_block(dg_ref) + b_ref[...]
    m = jnp.max(z, axis=-1, keepdims=True)
    e = jnp.exp(z - m)
    o_ref[...] = e / jnp.sum(e, axis=-1, keepdims=True)


_DG_SPEC = pl.BlockSpec((1, NC, BR), lambda i: (i, 0, 0))


def _tc_g(x, W, degp):
    return pl.pallas_call(
        _tc_g_body,
        grid=(GRID,),
        in_specs=[
            pl.BlockSpec((BR, D), lambda i: (i, 0)),
            pl.BlockSpec((D, D), lambda i: (0, 0)),
            _DG_SPEC,
        ],
        out_specs=pl.BlockSpec((BR, D), lambda i: (i, 0)),
        out_shape=jax.ShapeDtypeStruct((N, D), jnp.float32),
    )(x, W, degp)


def _tc_mid(p, g, degp, b_row, W):
    return pl.pallas_call(
        _tc_mid_body,
        grid=(GRID,),
        in_specs=[
            pl.BlockSpec((NC, BR, D), lambda i: (0, i, 0)),
            pl.BlockSpec((BR, D), lambda i: (i, 0)),
            _DG_SPEC,
            pl.BlockSpec((1, D), lambda i: (0, 0)),
            pl.BlockSpec((D, D), lambda i: (0, 0)),
        ],
        out_specs=pl.BlockSpec((BR, D), lambda i: (i, 0)),
        out_shape=jax.ShapeDtypeStruct((N, D), jnp.float32),
    )(p, g, degp, b_row, W)


def _tc_out(p, g, degp, b_row):
    return pl.pallas_call(
        _tc_out_body,
        grid=(GRID,),
        in_specs=[
            pl.BlockSpec((NC, BR, D), lambda i: (0, i, 0)),
            pl.BlockSpec((BR, D), lambda i: (i, 0)),
            _DG_SPEC,
            pl.BlockSpec((1, D), lambda i: (0, 0)),
        ],
        out_specs=pl.BlockSpec((BR, D), lambda i: (i, 0)),
        out_shape=jax.ShapeDtypeStruct((N, D), jnp.float32),
    )(p, g, degp, b_row)


def kernel(x, edge_index, W1, b1, W2, b2):
    ei4 = edge_index.reshape(2, NW, NCH, CH)
    zero2 = jnp.zeros((NPAD, D), jnp.float32)
    zero1 = jnp.zeros((RPT,), jnp.float32)

    degp = _sc_degree(ei4, zero1)[:, :N].reshape(NC, GRID, BR).transpose(1, 0, 2)
    g1 = _tc_g(x, W1, degp)
    p1 = _sc_aggregate(ei4, g1, zero2)
    g2 = _tc_mid(p1, g1, degp, b1.reshape(1, D), W2)
    p2 = _sc_aggregate(ei4, g2, zero2)
    return _tc_out(p2, g2, degp, b2.reshape(1, D))


# speedup vs baseline: 1.0586x; 1.0399x over previous
import functools

import jax
import jax.numpy as jnp
from jax import lax
from jax.experimental import pallas as pl
from jax.experimental.pallas import tpu as pltpu
from jax.experimental.pallas import tpu_sc as plsc

N = 10000
D = 128
E = 320000
NC = 2
NS = 16
NW = NC * NS
CH = 80
NCH = (E // NW) // CH
NPAD = 10240
RPT = NPAD // NS

_mesh = plsc.VectorSubcoreMesh(
    core_axis_name="c", subcore_axis_name="s", num_cores=NC, num_subcores=NS
)


@functools.partial(
    pl.kernel,
    out_type=jax.ShapeDtypeStruct((NC, NPAD), jnp.float32),
    mesh=_mesh,
    scratch_types=[
        pltpu.VMEM((NCH, CH), jnp.int32),
        pltpu.VMEM((CH,), jnp.float32),
        pltpu.VMEM((RPT,), jnp.float32),
        pltpu.SemaphoreType.DMA,
        pltpu.VMEM_SHARED((NPAD,), jnp.float32),
    ],
)
def _sc_degree(ei_hbm, out_hbm, dst_v, ones_v, zrow_v, sem, accum):
    c = lax.axis_index("c")
    s = lax.axis_index("s")
    wid = s * NC + c
    _offs = sorted(set(list(range(0, CH - 15, 16)) + [CH - 16]))
    for _o in _offs:
        ones_v[pl.ds(_o, 16)] = jnp.ones((16,), jnp.float32)
    for _o in range(0, RPT, 16):
        zrow_v[pl.ds(_o, 16)] = jnp.zeros((16,), jnp.float32)
    pltpu.sync_copy(ei_hbm.at[1, wid], dst_v)
    pltpu.sync_copy(zrow_v, accum.at[pl.ds(s * RPT, RPT)])
    plsc.subcore_barrier()

    LAG = 4

    def body(j, carry):
        pltpu.async_copy(ones_v, accum.at[dst_v.at[j]], sem, add=True)

        @pl.when(j >= LAG)
        def _():
            pltpu.make_async_copy(ones_v, accum.at[dst_v.at[j - LAG]], sem).wait()
        return carry

    lax.fori_loop(0, NCH, body, 0)
    for _t in range(LAG):
        pltpu.make_async_copy(ones_v, accum.at[dst_v.at[NCH - LAG + _t]], sem).wait()
    plsc.subcore_barrier()
    pltpu.sync_copy(accum.at[pl.ds(s * RPT, RPT)], out_hbm.at[c, pl.ds(s * RPT, RPT)])


@functools.partial(
    pl.kernel,
    out_type=jax.ShapeDtypeStruct((NC, NPAD, D), jnp.float32),
    mesh=_mesh,
    scratch_types=[
        pltpu.VMEM((64, CH), jnp.int32),
        pltpu.VMEM((64, CH), jnp.int32),
        pltpu.VMEM((CH, D), jnp.float32),
        pltpu.VMEM((CH, D), jnp.float32),
        pltpu.VMEM((CH, D), jnp.float32),
        pltpu.SemaphoreType.DMA,
        pltpu.SemaphoreType.DMA,
        pltpu.VMEM_SHARED((NPAD, D), jnp.float32),
    ],
)
def _sc_aggregate(ei_hbm, g_hbm, out_hbm,
                  src_v, dst_v, gbuf0, gbuf1, gbuf2, sem_g, sem_s, accum):
    c = lax.axis_index("c")
    s = lax.axis_index("s")
    wid = s * NC + c
    for _r in range(CH):
        for _o in range(D // 16):
            gbuf0[_r, pl.ds(_o * 16, 16)] = jnp.zeros((16,), jnp.float32)
    for t in range(RPT // CH):
        pltpu.async_copy(gbuf0, accum.at[pl.ds(s * RPT + t * CH, CH)], sem_g)
    pltpu.sync_copy(ei_hbm.at[0, wid, pl.ds(0, 64)], src_v)
    pltpu.sync_copy(ei_hbm.at[1, wid, pl.ds(0, 64)], dst_v)
    for t in range(RPT // CH):
        pltpu.make_async_copy(gbuf0, accum.at[pl.ds(s * RPT + t * CH, CH)], sem_g).wait()
    plsc.subcore_barrier()

    bufs = (gbuf0, gbuf1, gbuf2)

    def _wait_scatter(j, b):
        pltpu.make_async_copy(bufs[b], accum.at[dst_v.at[j]], sem_s).wait()

    for base, rows in ((0, 64), (64, NCH - 64)):
        if base > 0:
            pltpu.sync_copy(ei_hbm.at[0, wid, pl.ds(base, rows)], src_v.at[pl.ds(0, rows)])
            pltpu.sync_copy(ei_hbm.at[1, wid, pl.ds(base, rows)], dst_v.at[pl.ds(0, rows)])
        pltpu.async_copy(g_hbm.at[src_v.at[0]], bufs[0], sem_g)
        pltpu.async_copy(g_hbm.at[src_v.at[1]], bufs[1], sem_g)

        def body(k, carry):
            for b in range(3):
                j = 3 * k + b
                pltpu.make_async_copy(g_hbm.at[src_v.at[j]], bufs[b], sem_g).wait()
                pltpu.async_copy(bufs[b], accum.at[dst_v.at[j]], sem_s, add=True)

                @pl.when(j >= 1)
                def _():
                    _wait_scatter(j - 1, (b - 1) % 3)

                @pl.when(j + 2 < rows)
                def _():
                    pltpu.async_copy(g_hbm.at[src_v.at[j + 2]], bufs[(b + 2) % 3], sem_g)
            return carry

        lax.fori_loop(0, rows // 3, body, 0)
        for j in range(3 * (rows // 3), rows):
            b = j % 3
            pltpu.make_async_copy(g_hbm.at[src_v.at[j]], bufs[b], sem_g).wait()
            pltpu.async_copy(bufs[b], accum.at[dst_v.at[j]], sem_s, add=True)
            if j >= 1:
                _wait_scatter(j - 1, (b - 1) % 3)
            if j + 2 < rows:
                pltpu.async_copy(g_hbm.at[src_v.at[j + 2]], bufs[(b + 2) % 3], sem_g)
        _wait_scatter(rows - 1, (rows - 1) % 3)
    plsc.subcore_barrier()
    for t in range(RPT // 128):
        _r = s * RPT + t * 128
        pltpu.async_copy(accum.at[pl.ds(_r, 128)], out_hbm.at[c, pl.ds(_r, 128)], sem_g)
    for t in range(RPT // 128):
        _r = s * RPT + t * 128
        pltpu.make_async_copy(accum.at[pl.ds(_r, 128)], out_hbm.at[c, pl.ds(_r, 128)], sem_g).wait()


BR = 5000
GRID = N // BR


def _dinv_block(dg_ref):
    deg = dg_ref[0, 0] + dg_ref[0, 1] + 1.0
    return jax.lax.rsqrt(deg)[:, None]


def _tc_g_body(x_ref, w_ref, dg_ref, o_ref):
    h = jnp.dot(x_ref[...], w_ref[...], preferred_element_type=jnp.float32)
    o_ref[...] = h * _dinv_block(dg_ref)


def _tc_mid_body(p_ref, g_ref, dg_ref, b_ref, w_ref, o_ref):
    dinv = _dinv_block(dg_ref)
    z = (p_ref[0] + p_ref[1] + g_ref[...]) * dinv + b_ref[...]
    a = jax.nn.gelu(z)
    h2 = jnp.dot(a, w_ref[...], preferred_element_type=jnp.float32)
    o_ref[...] = h2 * dinv


def _tc_out_body(p_ref, g_ref, dg_ref, b_ref, o_ref):
    z = (p_ref[0] + p_ref[1] + g_ref[...]) * _dinv_block(dg_ref) + b_ref[...]
    m = jnp.max(z, axis=-1, keepdims=True)
    e = jnp.exp(z - m)
    o_ref[...] = e / jnp.sum(e, axis=-1, keepdims=True)


_DG_SPEC = pl.BlockSpec((1, NC, BR), lambda i: (i, 0, 0))


def _tc_g(x, W, degp):
    return pl.pallas_call(
        _tc_g_body,
        grid=(GRID,),
        in_specs=[
            pl.BlockSpec((BR, D), lambda i: (i, 0)),
            pl.BlockSpec((D, D), lambda i: (0, 0)),
            _DG_SPEC,
        ],
        out_specs=pl.BlockSpec((BR, D), lambda i: (i, 0)),
        out_shape=jax.ShapeDtypeStruct((N, D), jnp.float32),
    )(x, W, degp)


def _tc_mid(p, g, degp, b_row, W):
    return pl.pallas_call(
        _tc_mid_body,
        grid=(GRID,),
        in_specs=[
            pl.BlockSpec((NC, BR, D), lambda i: (0, i, 0)),
            pl.BlockSpec((BR, D), lambda i: (i, 0)),
            _DG_SPEC,
            pl.BlockSpec((1, D), lambda i: (0, 0)),
            pl.BlockSpec((D, D), lambda i: (0, 0)),
        ],
        out_specs=pl.BlockSpec((BR, D), lambda i: (i, 0)),
        out_shape=jax.ShapeDtypeStruct((N, D), jnp.float32),
    )(p, g, degp, b_row, W)


def _tc_out(p, g, degp, b_row):
    return pl.pallas_call(
        _tc_out_body,
        grid=(GRID,),
        in_specs=[
            pl.BlockSpec((NC, BR, D), lambda i: (0, i, 0)),
            pl.BlockSpec((BR, D), lambda i: (i, 0)),
            _DG_SPEC,
            pl.BlockSpec((1, D), lambda i: (0, 0)),
        ],
        out_specs=pl.BlockSpec((BR, D), lambda i: (i, 0)),
        out_shape=jax.ShapeDtypeStruct((N, D), jnp.float32),
    )(p, g, degp, b_row)


def kernel(x, edge_index, W1, b1, W2, b2):
    ei4 = edge_index.reshape(2, NW, NCH, CH)

    degp = _sc_degree(ei4)[:, :N].reshape(NC, GRID, BR).transpose(1, 0, 2)
    g1 = _tc_g(x, W1, degp)
    p1 = _sc_aggregate(ei4, g1)
    g2 = _tc_mid(p1, g1, degp, b1.reshape(1, D), W2)
    p2 = _sc_aggregate(ei4, g2)
    return _tc_out(p2, g2, degp, b2.reshape(1, D))
